# baseline (speedup 1.0000x reference)
import functools

import jax
import jax.numpy as jnp
from jax import lax
from jax.experimental import pallas as pl
from jax.experimental.pallas import tpu as pltpu
from jax.experimental.pallas import tpu_sc as plsc


def _build_sc_call(B, T, V, D, NB):
    info = plsc.get_sparse_core_info()
    NC, NS, L = info.num_cores, info.num_subcores, info.num_lanes
    NW = NC * NS
    assert B % NW == 0
    b_per_w = B // NW
    assert b_per_w % NB == 0
    CHUNK = NB * T
    assert CHUNK <= 128
    n_chunks = b_per_w // NB
    assert n_chunks % 2 == 0 and n_chunks >= 4
    lanes = D // L
    POS_PAD = ((T + 7) // 8) * 8

    mesh = plsc.VectorSubcoreMesh(core_axis_name="c", subcore_axis_name="s")

    @functools.partial(
        pl.kernel,
        mesh=mesh,
        compiler_params=pltpu.CompilerParams(use_tc_tiling_on_sc=True),
        out_type=jax.ShapeDtypeStruct((B, T, D), jnp.float32),
        scratch_types=[
            pltpu.VMEM((b_per_w * T,), jnp.int32),
            pltpu.VMEM((CHUNK, D), jnp.float32),
            pltpu.VMEM((CHUNK, D), jnp.float32),
            pltpu.VMEM((NB, T, D), jnp.float32),
            pltpu.VMEM((NB, T, D), jnp.float32),
            pltpu.VMEM((POS_PAD, D), jnp.float32),
            pltpu.SemaphoreType.DMA,
            pltpu.SemaphoreType.DMA,
            pltpu.SemaphoreType.DMA,
            pltpu.SemaphoreType.DMA,
            pltpu.SemaphoreType.DMA,
            pltpu.SemaphoreType.DMA,
        ],
    )
    def sc_kernel(table_hbm, idx_hbm, pos_hbm, out_hbm,
                  idx_v, gb0, gb1, ob0, ob1, pos_v,
                  gsa0, gsa1, gsb0, gsb1, os0, os1):
        wid = lax.axis_index("s") * NC + lax.axis_index("c")
        wrow = wid * b_per_w * T
        wb = wid * b_per_w
        H = CHUNK // 2
        gb, ob, osem = [gb0, gb1], [ob0, ob1], [os0, os1]
        gs = [[gsa0, gsa1], [gsb0, gsb1]]

        pltpu.sync_copy(pos_hbm.at[pl.ds(0, POS_PAD)], pos_v)
        pltpu.sync_copy(idx_hbm.at[pl.ds(wrow, b_per_w * T)], idx_v)

        def g_src(i, h):
            return table_hbm.at[
                idx_v.at[pl.ds(pl.multiple_of(i * CHUNK + h * H, H), H)]]

        def g_dst(s, h):
            return gb[s].at[pl.ds(h * H, H)]

        def g_start(i, s):
            for h in range(2):
                pltpu.async_copy(g_src(i, h), g_dst(s, h), gs[h][s])

        def g_wait(i, s):
            for h in range(2):
                pltpu.make_async_copy(g_src(i, h), g_dst(s, h), gs[h][s]).wait()

        def o_dst(i):
            return out_hbm.at[pl.ds(pl.multiple_of(wb + i * NB, NB), NB)]

        def add_chunk(s):
            def add_rows(t, carry):
                pv = [pos_v[t, pl.ds(c * L, L)] for c in range(lanes)]
                for nb in range(NB):
                    r = t + nb * T
                    for c in range(lanes):
                        sl = pl.ds(c * L, L)
                        ob[s][nb, t, sl] = gb[s][r, sl] + pv[c]
                return carry

            lax.fori_loop(0, T, add_rows, 0, unroll=2)

        def step(i, s, prefetch):
            g_wait(i, s)

            @pl.when(i >= 2)
            def _():
                pltpu.make_async_copy(ob[s], o_dst(i - 2), osem[s]).wait()

            add_chunk(s)
            if prefetch:
                g_start(i + 2, s)
            pltpu.async_copy(ob[s], o_dst(i), osem[s])

        for s in range(2):
            g_start(s, s)

        def outer(o, carry):
            for s in range(2):
                step(o * 2 + s, s, prefetch=True)
            return carry

        lax.fori_loop(0, n_chunks // 2 - 1, outer, 0, unroll=False)
        for s in range(2):
            step(n_chunks - 2 + s, s, prefetch=False)
        for s in range(2):
            pltpu.make_async_copy(ob[s], o_dst(n_chunks - 2 + s), osem[s]).wait()

    return sc_kernel


def kernel(move_tokens, token_table, pos_table):
    B, T = move_tokens.shape
    V, D = token_table.shape
    flat_idx = move_tokens.reshape(B * T).astype(jnp.int32)
    sc_call = _build_sc_call(B, T, V, D, NB=4)
    return sc_call(token_table, flat_idx, pos_table)


# speedup vs baseline: 1.2305x; 1.2305x over previous
import functools

import jax
import jax.numpy as jnp
from jax import lax
from jax.experimental import pallas as pl
from jax.experimental.pallas import tpu as pltpu
from jax.experimental.pallas import tpu_sc as plsc

_GDEPTH = 4
_ODEPTH = 2


def _build_sc_call(B, T, V, D, NB):
    info = plsc.get_sparse_core_info()
    NC, NS, L = info.num_cores, info.num_subcores, info.num_lanes
    NW = NC * NS
    assert B % NW == 0
    b_per_w = B // NW
    assert b_per_w % NB == 0
    CHUNK = NB * T
    assert CHUNK <= 128
    n_chunks = b_per_w // NB
    assert n_chunks % _GDEPTH == 0 and n_chunks >= 2 * _GDEPTH
    lanes = D // L
    POS_PAD = ((T + 7) // 8) * 8

    mesh = plsc.VectorSubcoreMesh(core_axis_name="c", subcore_axis_name="s")

    @functools.partial(
        pl.kernel,
        mesh=mesh,
        compiler_params=pltpu.CompilerParams(use_tc_tiling_on_sc=True),
        out_type=jax.ShapeDtypeStruct((B, T, D), jnp.float32),
        scratch_types=(
            [pltpu.VMEM((b_per_w * T,), jnp.int32)]
            + [pltpu.VMEM((CHUNK, D), jnp.float32) for _ in range(_GDEPTH)]
            + [pltpu.VMEM((NB, T, D), jnp.float32) for _ in range(_ODEPTH)]
            + [pltpu.VMEM((POS_PAD, D), jnp.float32)]
            + [pltpu.SemaphoreType.DMA] * (_GDEPTH + _ODEPTH)
        ),
    )
    def sc_kernel(table_hbm, idx_hbm, pos_hbm, out_hbm, idx_v, *bufs):
        gb = list(bufs[:_GDEPTH])
        ob = list(bufs[_GDEPTH:_GDEPTH + _ODEPTH])
        pos_v = bufs[_GDEPTH + _ODEPTH]
        gs = list(bufs[_GDEPTH + _ODEPTH + 1:_GDEPTH + _ODEPTH + 1 + _GDEPTH])
        osem = list(bufs[_GDEPTH + _ODEPTH + 1 + _GDEPTH:])

        wid = lax.axis_index("s") * NC + lax.axis_index("c")
        wrow = wid * b_per_w * T
        wb = wid * b_per_w

        pltpu.sync_copy(pos_hbm.at[pl.ds(0, POS_PAD)], pos_v)
        pltpu.sync_copy(idx_hbm.at[pl.ds(wrow, b_per_w * T)], idx_v)

        def g_src(i):
            return table_hbm.at[idx_v.at[pl.ds(pl.multiple_of(i * CHUNK, CHUNK), CHUNK)]]

        def o_dst(i):
            return out_hbm.at[pl.ds(pl.multiple_of(wb + i * NB, NB), NB)]

        def add_chunk(gk, ok):
            def add_rows(t, carry):
                pv = [pos_v[t, pl.ds(c * L, L)] for c in range(lanes)]
                for nb in range(NB):
                    r = t + nb * T
                    for c in range(lanes):
                        sl = pl.ds(c * L, L)
                        ob[ok][nb, t, sl] = gb[gk][r, sl] + pv[c]
                return carry

            lax.fori_loop(0, T, add_rows, 0, unroll=False)

        def step(i, gk, ok, prefetch):
            pltpu.make_async_copy(g_src(i), gb[gk], gs[gk]).wait()

            @pl.when(i >= _ODEPTH)
            def _():
                pltpu.make_async_copy(ob[ok], o_dst(i - _ODEPTH), osem[ok]).wait()

            add_chunk(gk, ok)
            if prefetch:
                pltpu.async_copy(g_src(i + _GDEPTH), gb[gk], gs[gk])
            pltpu.async_copy(ob[ok], o_dst(i), osem[ok])

        for u in range(_GDEPTH):
            pltpu.async_copy(g_src(u), gb[u], gs[u])

        def outer(r, carry):
            for u in range(_GDEPTH):
                i = r * _GDEPTH + u
                step(i, u, u % _ODEPTH, prefetch=True)
            return carry

        lax.fori_loop(0, n_chunks // _GDEPTH - 1, outer, 0, unroll=False)
        for u in range(_GDEPTH):
            i = n_chunks - _GDEPTH + u
            step(i, u, u % _ODEPTH, prefetch=False)
        for u in range(_ODEPTH):
            i = n_chunks - _ODEPTH + u
            pltpu.make_async_copy(ob[u % _ODEPTH], o_dst(i), osem[u % _ODEPTH]).wait()

    return sc_kernel


def kernel(move_tokens, token_table, pos_table):
    B, T = move_tokens.shape
    V, D = token_table.shape
    flat_idx = move_tokens.reshape(B * T).astype(jnp.int32)
    sc_call = _build_sc_call(B, T, V, D, NB=2)
    return sc_call(token_table, flat_idx, pos_table)
